# baseline (speedup 1.0000x reference)
import numpy as np
import jax
import jax.numpy as jnp
from jax.experimental import pallas as pl
from jax.experimental.pallas import tpu as pltpu

P = 22
RADIUS = 2.0
L = 128
NSTEPS = 10
GSUB = 8
H = 1


def _ln(x, g, b):
    m = jnp.mean(x, axis=-1, keepdims=True)
    d = x - m
    v = jnp.mean(d * d, axis=-1, keepdims=True)
    return d * jax.lax.rsqrt(v + 1e-5) * g + b


def _dot(a, b):
    return jnp.dot(a, b, preferred_element_type=jnp.float32)


def _body(pos_ref, w, out_ref):
    w = jax.tree.map(lambda r: r[...], w)
    R = P * GSUB
    E2 = P * P * GSUB
    hs = range(H)

    pos = [pos_ref[pl.ds(k * R, R), :] for k in hs]
    mr = [pos[k][:, 15:18] for k in hs]
    nf = [pos[k][:, 3:18] - pos[k][:, 0:15] for k in hs]

    t = [jnp.maximum(_dot(nf[k], w["enW0"]) + w["enb0"], 0.0) for k in hs]
    t = [jnp.maximum(_dot(t[k], w["enW1"]) + w["enb1"], 0.0) for k in hs]
    x = [_dot(t[k], w["enW2"]) + w["enb2"] for k in hs]
    x = [_ln(x[k], w["enlg"], w["enlb"]) for k in hs]

    disp = [mr[k].reshape(P, 1, GSUB, 3) - mr[k].reshape(1, P, GSUB, 3) for k in hs]
    dist = [jnp.sqrt(jnp.sum(disp[k] * disp[k], axis=-1, keepdims=True)) for k in hs]
    ef = [jnp.concatenate([disp[k], dist[k]], axis=-1).reshape(E2, 4) for k in hs]
    mask = [(dist[k] <= RADIUS).astype(jnp.float32).reshape(E2, 1) for k in hs]

    t = [jnp.maximum(_dot(ef[k], w["eeW0"]) + w["eeb0"], 0.0) for k in hs]
    t = [jnp.maximum(_dot(t[k], w["eeW1"]) + w["eeb1"], 0.0) for k in hs]
    e = [_dot(t[k], w["eeW2"]) + w["eeb2"] for k in hs]
    e = [_ln(e[k], w["eelg"], w["eelb"]) for k in hs]

    for s in range(NSTEPS):
        xs = [_dot(x[k], w["pW1s"][s]) + w["pb1"][s] for k in hs]
        xr = [_dot(x[k], w["pW1r"][s]) for k in hs]
        t = [_dot(e[k], w["pW1e"][s]).reshape(P, P, GSUB, L) for k in hs]
        h1 = [jnp.maximum(
            t[k] + xs[k].reshape(P, 1, GSUB, L) + xr[k].reshape(1, P, GSUB, L),
            0.0).reshape(E2, L) for k in hs]
        h2 = [jnp.maximum(_dot(h1[k], w["pW2"][s]) + w["pb2"][s], 0.0) for k in hs]
        en = [_dot(h2[k], w["pW3"][s]) + w["pb3"][s] for k in hs]
        en = [_ln(en[k], w["pelg"][s], w["pelb"][s]) for k in hs]
        agg = [jnp.sum((en[k] * mask[k]).reshape(P, R, L), axis=0) for k in hs]
        t = [jnp.maximum(_dot(x[k], w["nW1x"][s]) + _dot(agg[k], w["nW1a"][s])
                         + w["nb1"][s], 0.0) for k in hs]
        t = [jnp.maximum(_dot(t[k], w["nW2"][s]) + w["nb2"][s], 0.0) for k in hs]
        nn = [_dot(t[k], w["nW3"][s]) + w["nb3"][s] for k in hs]
        nn = [_ln(nn[k], w["nlg"][s], w["nlb"][s]) for k in hs]
        x = [x[k] + nn[k] for k in hs]
        e = [e[k] + en[k] for k in hs]

    t = [jnp.maximum(_dot(x[k], w["dW0"]) + w["db0"], 0.0) for k in hs]
    t = [jnp.maximum(_dot(t[k], w["dW1"]) + w["db1"], 0.0) for k in hs]
    vel = [_dot(t[k], w["dW2"]) + w["db2"] for k in hs]
    for k in hs:
        out_ref[pl.ds(k * R, R), :] = mr[k] + vel[k]


def kernel(current_positions, params):
    cp = current_positions
    N = cp.shape[0]
    B = N // P
    NBLK = B // GSUB
    NB = NBLK // H
    R = P * GSUB

    posf = cp.reshape(N, 18)
    pos_t = posf.reshape(NBLK, GSUB, P, 18).transpose(0, 2, 1, 3).reshape(NBLK * R, 18)

    def stk(f):
        return jnp.stack([f(sp) for sp in params["proc"]])

    enc_n = params["enc_node_mlp"]
    enc_e = params["enc_edge_mlp"]
    dec = params["dec_mlp"]
    w = {
        "enW0": enc_n[0]["W"], "enb0": enc_n[0]["b"][None],
        "enW1": enc_n[1]["W"], "enb1": enc_n[1]["b"][None],
        "enW2": enc_n[2]["W"], "enb2": enc_n[2]["b"][None],
        "enlg": params["enc_node_ln"]["g"][None], "enlb": params["enc_node_ln"]["b"][None],
        "eeW0": enc_e[0]["W"], "eeb0": enc_e[0]["b"][None],
        "eeW1": enc_e[1]["W"], "eeb1": enc_e[1]["b"][None],
        "eeW2": enc_e[2]["W"], "eeb2": enc_e[2]["b"][None],
        "eelg": params["enc_edge_ln"]["g"][None], "eelb": params["enc_edge_ln"]["b"][None],
        "pW1r": stk(lambda sp: sp["edge_mlp"][0]["W"][:L]),
        "pW1s": stk(lambda sp: sp["edge_mlp"][0]["W"][L:2 * L]),
        "pW1e": stk(lambda sp: sp["edge_mlp"][0]["W"][2 * L:]),
        "pb1": stk(lambda sp: sp["edge_mlp"][0]["b"][None]),
        "pW2": stk(lambda sp: sp["edge_mlp"][1]["W"]),
        "pb2": stk(lambda sp: sp["edge_mlp"][1]["b"][None]),
        "pW3": stk(lambda sp: sp["edge_mlp"][2]["W"]),
        "pb3": stk(lambda sp: sp["edge_mlp"][2]["b"][None]),
        "pelg": stk(lambda sp: sp["edge_ln"]["g"][None]),
        "pelb": stk(lambda sp: sp["edge_ln"]["b"][None]),
        "nW1x": stk(lambda sp: sp["node_mlp"][0]["W"][:L]),
        "nW1a": stk(lambda sp: sp["node_mlp"][0]["W"][L:]),
        "nb1": stk(lambda sp: sp["node_mlp"][0]["b"][None]),
        "nW2": stk(lambda sp: sp["node_mlp"][1]["W"]),
        "nb2": stk(lambda sp: sp["node_mlp"][1]["b"][None]),
        "nW3": stk(lambda sp: sp["node_mlp"][2]["W"]),
        "nb3": stk(lambda sp: sp["node_mlp"][2]["b"][None]),
        "nlg": stk(lambda sp: sp["node_ln"]["g"][None]),
        "nlb": stk(lambda sp: sp["node_ln"]["b"][None]),
        "dW0": dec[0]["W"], "db0": dec[0]["b"][None],
        "dW1": dec[1]["W"], "db1": dec[1]["b"][None],
        "dW2": dec[2]["W"], "db2": dec[2]["b"][None],
    }

    pos_spec = pl.BlockSpec((H * R, 18), lambda g: (g, 0))
    w_specs = jax.tree.map(
        lambda a: pl.BlockSpec(a.shape, lambda g, nd=a.ndim: (0,) * nd), w
    )

    out_t = pl.pallas_call(
        _body,
        grid=(NB,),
        in_specs=(pos_spec, w_specs),
        out_specs=pl.BlockSpec((H * R, 3), lambda g: (g, 0)),
        out_shape=jax.ShapeDtypeStruct((NBLK * R, 3), jnp.float32),
        compiler_params=pltpu.CompilerParams(
            dimension_semantics=("parallel",),
        ),
    )(pos_t, w)

    out = out_t.reshape(NBLK, P, GSUB, 3).transpose(0, 2, 1, 3).reshape(N, 3)
    return out


# speedup vs baseline: 1.0874x; 1.0874x over previous
import numpy as np
import jax
import jax.numpy as jnp
from jax.experimental import pallas as pl
from jax.experimental.pallas import tpu as pltpu

P = 22
RADIUS = 2.0
L = 128
NSTEPS = 10
GSUB = 8
H = 2


def _ln(x, g, b):
    m = jnp.mean(x, axis=-1, keepdims=True)
    d = x - m
    v = jnp.mean(d * d, axis=-1, keepdims=True)
    return d * jax.lax.rsqrt(v + 1e-5) * g + b


def _dot(a, b):
    return jnp.dot(a, b, preferred_element_type=jnp.float32)


def _ln_j(x, g, b, J):
    m = _dot(x, J)
    d = x - m
    v = _dot(d * d, J)
    return d * jax.lax.rsqrt(v + 1e-5) * g + b


def _body(pos_ref, w, out_ref):
    w = jax.tree.map(lambda r: r[...], w)
    R = P * GSUB
    E2 = P * P * GSUB
    hs = range(H)

    pos = [pos_ref[pl.ds(k * R, R), :] for k in hs]
    mr = [pos[k][:, 15:18] for k in hs]
    nf = [pos[k][:, 3:18] - pos[k][:, 0:15] for k in hs]

    t = [jnp.maximum(_dot(nf[k], w["enW0"]) + w["enb0"], 0.0) for k in hs]
    t = [jnp.maximum(_dot(t[k], w["enW1"]) + w["enb1"], 0.0) for k in hs]
    x = [_dot(t[k], w["enW2"]) + w["enb2"] for k in hs]
    x = [_ln(x[k], w["enlg"], w["enlb"]) for k in hs]

    disp = [mr[k].reshape(P, 1, GSUB, 3) - mr[k].reshape(1, P, GSUB, 3) for k in hs]
    dist = [jnp.sqrt(jnp.sum(disp[k] * disp[k], axis=-1, keepdims=True)) for k in hs]
    ef = [jnp.concatenate([disp[k], dist[k]], axis=-1).reshape(E2, 4) for k in hs]
    mask = [(dist[k] <= RADIUS).astype(jnp.float32).reshape(E2, 1) for k in hs]

    t = [jnp.maximum(_dot(ef[k], w["eeW0"]) + w["eeb0"], 0.0) for k in hs]
    t = [jnp.maximum(_dot(t[k], w["eeW1"]) + w["eeb1"], 0.0) for k in hs]
    e = [_dot(t[k], w["eeW2"]) + w["eeb2"] for k in hs]
    e = [_ln(e[k], w["eelg"], w["eelb"]) for k in hs]

    for s in range(NSTEPS):
        xs = [_dot(x[k], w["pW1s"][s]) + w["pb1"][s] for k in hs]
        xr = [_dot(x[k], w["pW1r"][s]) for k in hs]
        t = [_dot(e[k], w["pW1e"][s]).reshape(P, P, GSUB, L) for k in hs]
        h1 = [jnp.maximum(
            t[k] + xs[k].reshape(P, 1, GSUB, L) + xr[k].reshape(1, P, GSUB, L),
            0.0).reshape(E2, L) for k in hs]
        h2 = [jnp.maximum(_dot(h1[k], w["pW2"][s]) + w["pb2"][s], 0.0) for k in hs]
        en = [_dot(h2[k], w["pW3"][s]) + w["pb3"][s] for k in hs]
        en = [_ln(en[0], w["pelg"][s], w["pelb"][s]),
              _ln_j(en[1], w["pelg"][s], w["pelb"][s], w["J"])]
        agg = [jnp.sum((en[k] * mask[k]).reshape(P, R, L), axis=0) for k in hs]
        t = [jnp.maximum(_dot(x[k], w["nW1x"][s]) + _dot(agg[k], w["nW1a"][s])
                         + w["nb1"][s], 0.0) for k in hs]
        t = [jnp.maximum(_dot(t[k], w["nW2"][s]) + w["nb2"][s], 0.0) for k in hs]
        nn = [_dot(t[k], w["nW3"][s]) + w["nb3"][s] for k in hs]
        nn = [_ln(nn[k], w["nlg"][s], w["nlb"][s]) for k in hs]
        x = [x[k] + nn[k] for k in hs]
        e = [e[k] + en[k] for k in hs]

    t = [jnp.maximum(_dot(x[k], w["dW0"]) + w["db0"], 0.0) for k in hs]
    t = [jnp.maximum(_dot(t[k], w["dW1"]) + w["db1"], 0.0) for k in hs]
    vel = [_dot(t[k], w["dW2"]) + w["db2"] for k in hs]
    for k in hs:
        out_ref[pl.ds(k * R, R), :] = mr[k] + vel[k]


def kernel(current_positions, params):
    cp = current_positions
    N = cp.shape[0]
    B = N // P
    NBLK = B // GSUB
    NB = NBLK // H
    R = P * GSUB

    posf = cp.reshape(N, 18)
    pos_t = posf.reshape(NBLK, GSUB, P, 18).transpose(0, 2, 1, 3).reshape(NBLK * R, 18)

    def stk(f):
        return jnp.stack([f(sp) for sp in params["proc"]])

    enc_n = params["enc_node_mlp"]
    enc_e = params["enc_edge_mlp"]
    dec = params["dec_mlp"]
    w = {
        "enW0": enc_n[0]["W"], "enb0": enc_n[0]["b"][None],
        "enW1": enc_n[1]["W"], "enb1": enc_n[1]["b"][None],
        "enW2": enc_n[2]["W"], "enb2": enc_n[2]["b"][None],
        "enlg": params["enc_node_ln"]["g"][None], "enlb": params["enc_node_ln"]["b"][None],
        "eeW0": enc_e[0]["W"], "eeb0": enc_e[0]["b"][None],
        "eeW1": enc_e[1]["W"], "eeb1": enc_e[1]["b"][None],
        "eeW2": enc_e[2]["W"], "eeb2": enc_e[2]["b"][None],
        "eelg": params["enc_edge_ln"]["g"][None], "eelb": params["enc_edge_ln"]["b"][None],
        "pW1r": stk(lambda sp: sp["edge_mlp"][0]["W"][:L]),
        "pW1s": stk(lambda sp: sp["edge_mlp"][0]["W"][L:2 * L]),
        "pW1e": stk(lambda sp: sp["edge_mlp"][0]["W"][2 * L:]),
        "pb1": stk(lambda sp: sp["edge_mlp"][0]["b"][None]),
        "pW2": stk(lambda sp: sp["edge_mlp"][1]["W"]),
        "pb2": stk(lambda sp: sp["edge_mlp"][1]["b"][None]),
        "pW3": stk(lambda sp: sp["edge_mlp"][2]["W"]),
        "pb3": stk(lambda sp: sp["edge_mlp"][2]["b"][None]),
        "pelg": stk(lambda sp: sp["edge_ln"]["g"][None]),
        "pelb": stk(lambda sp: sp["edge_ln"]["b"][None]),
        "nW1x": stk(lambda sp: sp["node_mlp"][0]["W"][:L]),
        "nW1a": stk(lambda sp: sp["node_mlp"][0]["W"][L:]),
        "nb1": stk(lambda sp: sp["node_mlp"][0]["b"][None]),
        "nW2": stk(lambda sp: sp["node_mlp"][1]["W"]),
        "nb2": stk(lambda sp: sp["node_mlp"][1]["b"][None]),
        "nW3": stk(lambda sp: sp["node_mlp"][2]["W"]),
        "nb3": stk(lambda sp: sp["node_mlp"][2]["b"][None]),
        "nlg": stk(lambda sp: sp["node_ln"]["g"][None]),
        "nlb": stk(lambda sp: sp["node_ln"]["b"][None]),
        "J": jnp.full((L, L), 1.0 / L, jnp.float32),
        "dW0": dec[0]["W"], "db0": dec[0]["b"][None],
        "dW1": dec[1]["W"], "db1": dec[1]["b"][None],
        "dW2": dec[2]["W"], "db2": dec[2]["b"][None],
    }

    pos_spec = pl.BlockSpec((H * R, 18), lambda g: (g, 0))
    w_specs = jax.tree.map(
        lambda a: pl.BlockSpec(a.shape, lambda g, nd=a.ndim: (0,) * nd), w
    )

    out_t = pl.pallas_call(
        _body,
        grid=(NB,),
        in_specs=(pos_spec, w_specs),
        out_specs=pl.BlockSpec((H * R, 3), lambda g: (g, 0)),
        out_shape=jax.ShapeDtypeStruct((NBLK * R, 3), jnp.float32),
        compiler_params=pltpu.CompilerParams(
            dimension_semantics=("parallel",),
        ),
    )(pos_t, w)

    out = out_t.reshape(NBLK, P, GSUB, 3).transpose(0, 2, 1, 3).reshape(N, 3)
    return out


# speedup vs baseline: 1.1947x; 1.0987x over previous
import numpy as np
import jax
import jax.numpy as jnp
from jax.experimental import pallas as pl
from jax.experimental.pallas import tpu as pltpu

P = 22
RADIUS = 2.0
L = 128
NSTEPS = 10
GSUB = 8
H = 2


def _ln(x, g, b):
    m = jnp.mean(x, axis=-1, keepdims=True)
    d = x - m
    v = jnp.mean(d * d, axis=-1, keepdims=True)
    return d * jax.lax.rsqrt(v + 1e-5) * g + b


def _dot(a, b):
    return jnp.dot(a, b, preferred_element_type=jnp.float32)


def _body(pos_ref, w, out_ref):
    w = jax.tree.map(lambda r: r[...], w)
    R = P * GSUB
    E2 = P * P * GSUB
    hs = range(H)

    pos = [pos_ref[pl.ds(k * R, R), :] for k in hs]
    mr = [pos[k][:, 15:18] for k in hs]
    nf = [pos[k][:, 3:18] - pos[k][:, 0:15] for k in hs]

    t = [jnp.maximum(_dot(nf[k], w["enW0"]) + w["enb0"], 0.0) for k in hs]
    t = [jnp.maximum(_dot(t[k], w["enW1"]) + w["enb1"], 0.0) for k in hs]
    x = [_dot(t[k], w["enW2"]) + w["enb2"] for k in hs]
    x = [_ln(x[k], w["enlg"], w["enlb"]) for k in hs]

    disp = [mr[k].reshape(P, 1, GSUB, 3) - mr[k].reshape(1, P, GSUB, 3) for k in hs]
    dist = [jnp.sqrt(jnp.sum(disp[k] * disp[k], axis=-1, keepdims=True)) for k in hs]
    ef = [jnp.concatenate([disp[k], dist[k]], axis=-1).reshape(E2, 4) for k in hs]
    mask = [(dist[k] <= RADIUS).astype(jnp.float32).reshape(E2, 1) for k in hs]

    t = [jnp.maximum(_dot(ef[k], w["eeW0"]) + w["eeb0"], 0.0) for k in hs]
    t = [jnp.maximum(_dot(t[k], w["eeW1"]) + w["eeb1"], 0.0) for k in hs]
    e = [_dot(t[k], w["eeW2"]) + w["eeb2"] for k in hs]
    e = [_ln(e[k], w["eelg"], w["eelb"]) for k in hs]

    for s in range(NSTEPS):
        xs = [_dot(x[k], w["pW1s"][s]) + w["pb1"][s] for k in hs]
        xr = [_dot(x[k], w["pW1r"][s]) for k in hs]
        t = [_dot(e[k], w["pW1e"][s]).reshape(P, P, GSUB, L) for k in hs]
        h1 = [jnp.maximum(
            t[k] + xs[k].reshape(P, 1, GSUB, L) + xr[k].reshape(1, P, GSUB, L),
            0.0).reshape(E2, L) for k in hs]
        h2 = [jnp.maximum(_dot(h1[k], w["pW2"][s]) + w["pb2"][s], 0.0) for k in hs]
        en = [_dot(h2[k], w["pW3"][s]) + w["pb3"][s] for k in hs]
        en = [_ln(en[k], w["pelg"][s], w["pelb"][s]) for k in hs]
        agg = [jnp.sum((en[k] * mask[k]).reshape(P, R, L), axis=0) for k in hs]
        t = [jnp.maximum(_dot(x[k], w["nW1x"][s]) + _dot(agg[k], w["nW1a"][s])
                         + w["nb1"][s], 0.0) for k in hs]
        t = [jnp.maximum(_dot(t[k], w["nW2"][s]) + w["nb2"][s], 0.0) for k in hs]
        nn = [_dot(t[k], w["nW3"][s]) + w["nb3"][s] for k in hs]
        nn = [_ln(nn[k], w["nlg"][s], w["nlb"][s]) for k in hs]
        x = [x[k] + nn[k] for k in hs]
        e = [e[k] + en[k] for k in hs]

    t = [jnp.maximum(_dot(x[k], w["dW0"]) + w["db0"], 0.0) for k in hs]
    t = [jnp.maximum(_dot(t[k], w["dW1"]) + w["db1"], 0.0) for k in hs]
    vel = [_dot(t[k], w["dW2"]) + w["db2"] for k in hs]
    for k in hs:
        out_ref[pl.ds(k * R, R), :] = mr[k] + vel[k]


def kernel(current_positions, params):
    cp = current_positions
    N = cp.shape[0]
    B = N // P
    NBLK = B // GSUB
    NB = NBLK // H
    R = P * GSUB

    posf = cp.reshape(N, 18)
    pos_t = posf.reshape(NBLK, GSUB, P, 18).transpose(0, 2, 1, 3).reshape(NBLK * R, 18)

    def stk(f):
        return jnp.stack([f(sp) for sp in params["proc"]])

    enc_n = params["enc_node_mlp"]
    enc_e = params["enc_edge_mlp"]
    dec = params["dec_mlp"]
    w = {
        "enW0": enc_n[0]["W"], "enb0": enc_n[0]["b"][None],
        "enW1": enc_n[1]["W"], "enb1": enc_n[1]["b"][None],
        "enW2": enc_n[2]["W"], "enb2": enc_n[2]["b"][None],
        "enlg": params["enc_node_ln"]["g"][None], "enlb": params["enc_node_ln"]["b"][None],
        "eeW0": enc_e[0]["W"], "eeb0": enc_e[0]["b"][None],
        "eeW1": enc_e[1]["W"], "eeb1": enc_e[1]["b"][None],
        "eeW2": enc_e[2]["W"], "eeb2": enc_e[2]["b"][None],
        "eelg": params["enc_edge_ln"]["g"][None], "eelb": params["enc_edge_ln"]["b"][None],
        "pW1r": stk(lambda sp: sp["edge_mlp"][0]["W"][:L]),
        "pW1s": stk(lambda sp: sp["edge_mlp"][0]["W"][L:2 * L]),
        "pW1e": stk(lambda sp: sp["edge_mlp"][0]["W"][2 * L:]),
        "pb1": stk(lambda sp: sp["edge_mlp"][0]["b"][None]),
        "pW2": stk(lambda sp: sp["edge_mlp"][1]["W"]),
        "pb2": stk(lambda sp: sp["edge_mlp"][1]["b"][None]),
        "pW3": stk(lambda sp: sp["edge_mlp"][2]["W"]),
        "pb3": stk(lambda sp: sp["edge_mlp"][2]["b"][None]),
        "pelg": stk(lambda sp: sp["edge_ln"]["g"][None]),
        "pelb": stk(lambda sp: sp["edge_ln"]["b"][None]),
        "nW1x": stk(lambda sp: sp["node_mlp"][0]["W"][:L]),
        "nW1a": stk(lambda sp: sp["node_mlp"][0]["W"][L:]),
        "nb1": stk(lambda sp: sp["node_mlp"][0]["b"][None]),
        "nW2": stk(lambda sp: sp["node_mlp"][1]["W"]),
        "nb2": stk(lambda sp: sp["node_mlp"][1]["b"][None]),
        "nW3": stk(lambda sp: sp["node_mlp"][2]["W"]),
        "nb3": stk(lambda sp: sp["node_mlp"][2]["b"][None]),
        "nlg": stk(lambda sp: sp["node_ln"]["g"][None]),
        "nlb": stk(lambda sp: sp["node_ln"]["b"][None]),
        "dW0": dec[0]["W"], "db0": dec[0]["b"][None],
        "dW1": dec[1]["W"], "db1": dec[1]["b"][None],
        "dW2": dec[2]["W"], "db2": dec[2]["b"][None],
    }

    pos_spec = pl.BlockSpec((H * R, 18), lambda g: (g, 0))
    w_specs = jax.tree.map(
        lambda a: pl.BlockSpec(a.shape, lambda g, nd=a.ndim: (0,) * nd), w
    )

    out_t = pl.pallas_call(
        _body,
        grid=(NB,),
        in_specs=(pos_spec, w_specs),
        out_specs=pl.BlockSpec((H * R, 3), lambda g: (g, 0)),
        out_shape=jax.ShapeDtypeStruct((NBLK * R, 3), jnp.float32),
        compiler_params=pltpu.CompilerParams(
            dimension_semantics=("parallel",),
        ),
    )(pos_t, w)

    out = out_t.reshape(NBLK, P, GSUB, 3).transpose(0, 2, 1, 3).reshape(N, 3)
    return out
